# Initial kernel scaffold; baseline (speedup 1.0000x reference)
import jax
import jax.numpy as jnp
from jax.experimental import pallas as pl


def kernel(coords, values):
    raise NotImplementedError("write your pallas kernel here")


# speedup vs baseline: 6.4321x; 6.4321x over previous
import functools

import jax
import jax.numpy as jnp
from jax import lax
from jax.experimental import pallas as pl
from jax.experimental.pallas import tpu as pltpu
from jax.experimental.pallas import tpu_sc as plsc

_B = 16
_N = 4096
_S = 2048
_C = 3
_D = 128
_L = 16
_NCH = _N // _L
_SCH = _S // _L


def _fps_body(coords2d, values2d, f0_hbm, outc2d, outv2d,
              xv, yv, zv, dist_v, idx_v, f0v, rowv, outrow):
    c = lax.axis_index("c")
    s = lax.axis_index("s")
    wid = s * 2 + c
    lane = lax.iota(jnp.int32, _L)

    @pl.when(wid < _B)
    def _():
        b = wid
        pltpu.sync_copy(coords2d.at[_C * b + 0], xv)
        pltpu.sync_copy(coords2d.at[_C * b + 1], yv)
        pltpu.sync_copy(coords2d.at[_C * b + 2], zv)
        pltpu.sync_copy(f0_hbm.at[b], f0v)
        fstart = f0v[...].astype(jnp.int32)

        def init(j, carry):
            dist_v[pl.ds(j * _L, _L)] = jnp.full((_L,), 1e10, jnp.float32)
            return carry

        lax.fori_loop(0, _NCH, init, 0)

        lane_f = lane.astype(jnp.float32)

        def fps_step(i, fvec):
            cx = plsc.load_gather(xv, [fvec])
            cy = plsc.load_gather(yv, [fvec])
            cz = plsc.load_gather(zv, [fvec])
            plsc.store_scatter(idx_v, [jnp.full((_L,), i, jnp.int32)],
                               fvec, mask=lane == 0)

            def sweep(j, carry):
                m, mi = carry
                sl = pl.ds(j * _L, _L)
                dx = xv[sl] - cx
                dy = yv[sl] - cy
                dz = zv[sl] - cz
                d = (dx * dx + dy * dy) + dz * dz
                dd = dist_v[sl]
                dn = jnp.where(d < dd, d, dd)
                dist_v[sl] = dn
                cond = dn > m
                m = jnp.where(cond, dn, m)
                mi = jnp.where(cond, (j * _L).astype(jnp.float32) + lane_f,
                               mi)
                return m, mi

            m, mi = lax.fori_loop(
                0, _NCH, sweep,
                (jnp.full((_L,), -1.0, jnp.float32),
                 jnp.full((_L,), 0.0, jnp.float32)))
            gmax = jnp.max(m)
            cand = jnp.where(m == gmax, mi, jnp.float32(1e9))
            fnew = jnp.min(cand)
            return jnp.full((_L,), fnew, jnp.float32).astype(jnp.int32)

        lax.fori_loop(0, _S, fps_step, fstart)

        def gather_value_row(d, carry):
            pltpu.sync_copy(values2d.at[_D * b + d], rowv)

            def g16(j, inner):
                sl = pl.ds(j * _L, _L)
                outrow[sl] = plsc.load_gather(rowv, [idx_v[sl]])
                return inner

            lax.fori_loop(0, _SCH, g16, 0)
            pltpu.sync_copy(outrow, outv2d.at[_D * b + d])
            return carry

        lax.fori_loop(0, _D, gather_value_row, 0)

        for ch, src in enumerate((xv, yv, zv)):
            def g16c(j, inner, src=src):
                sl = pl.ds(j * _L, _L)
                outrow[sl] = plsc.load_gather(src, [idx_v[sl]])
                return inner

            lax.fori_loop(0, _SCH, g16c, 0)
            pltpu.sync_copy(outrow, outc2d.at[_C * b + ch])


@jax.jit
def _run(coords2d, values2d, f0):
    mesh = plsc.VectorSubcoreMesh(core_axis_name="c", subcore_axis_name="s")
    return pl.kernel(
        _fps_body,
        out_type=(
            jax.ShapeDtypeStruct((_B * _C, _S), jnp.float32),
            jax.ShapeDtypeStruct((_B * _D, _S), jnp.float32),
        ),
        mesh=mesh,
        scratch_types=[
            pltpu.VMEM((_N,), jnp.float32),
            pltpu.VMEM((_N,), jnp.float32),
            pltpu.VMEM((_N,), jnp.float32),
            pltpu.VMEM((_N,), jnp.float32),
            pltpu.VMEM((_S,), jnp.int32),
            pltpu.VMEM((_L,), jnp.float32),
            pltpu.VMEM((_N,), jnp.float32),
            pltpu.VMEM((_S,), jnp.float32),
        ],
        compiler_params=pltpu.CompilerParams(needs_layout_passes=False),
        name="fps_subsample_sc",
    )(coords2d, values2d, f0)


def kernel(coords, values):
    B, C, N = coords.shape
    _, D, _ = values.shape
    S = _S
    f0 = jax.random.randint(jax.random.key(42), (B,), 0, N).astype(jnp.float32)
    f0 = jnp.broadcast_to(f0[:, None], (B, 16))
    outc2d, outv2d = _run(coords.reshape(B * C, N),
                          values.reshape(B * D, N), f0)
    return outc2d.reshape(B, C, S), outv2d.reshape(B, D, S)


# speedup vs baseline: 20.9431x; 3.2560x over previous
import functools

import jax
import jax.numpy as jnp
from jax import lax
from jax.experimental import pallas as pl
from jax.experimental.pallas import tpu as pltpu
from jax.experimental.pallas import tpu_sc as plsc

_B = 16
_N = 4096
_S = 2048
_C = 3
_D = 128
_L = 16
_NCH = _N // _L
_SCH = _S // _L


def _fps_body(coords2d, values2d, f0_hbm, outc2d, outv2d,
              xv, yv, zv, dist_v, idx_v, f0v, rowv, outrow):
    c = lax.axis_index("c")
    s = lax.axis_index("s")
    wid = s * 2 + c
    lane = lax.iota(jnp.int32, _L)

    @pl.when(wid < _B)
    def _():
        b = wid
        pltpu.sync_copy(coords2d.at[_C * b + 0], xv)
        pltpu.sync_copy(coords2d.at[_C * b + 1], yv)
        pltpu.sync_copy(coords2d.at[_C * b + 2], zv)
        pltpu.sync_copy(f0_hbm.at[b], f0v)
        fstart = f0v[...].astype(jnp.int32)

        @plsc.parallel_loop(0, _NCH, 1, unroll=8)
        def _init(j):
            dist_v[pl.ds(j * _L, _L)] = jnp.full((_L,), 1e10, jnp.float32)

        lane_f = lane.astype(jnp.float32)

        def fps_step(i, fvec):
            cx = plsc.load_gather(xv, [fvec])
            cy = plsc.load_gather(yv, [fvec])
            cz = plsc.load_gather(zv, [fvec])
            plsc.store_scatter(idx_v, [jnp.full((_L,), i, jnp.int32)],
                               fvec, mask=lane == 0)

            acc0 = (jnp.full((_L,), -1.0, jnp.float32),
                    jnp.full((_L,), 0.0, jnp.float32))

            @plsc.parallel_loop(0, _NCH, 1, unroll=8, carry=acc0)
            def sweep(j, carry):
                m, mi = carry
                sl = pl.ds(j * _L, _L)
                dx = xv[sl] - cx
                dy = yv[sl] - cy
                dz = zv[sl] - cz
                d = (dx * dx + dz * dz) + dy * dy
                dd = dist_v[sl]
                dn = jnp.where(d < dd, d, dd)
                dist_v[sl] = dn
                cond = dn > m
                m = jnp.where(cond, dn, m)
                mi = jnp.where(cond, (j * _L).astype(jnp.float32) + lane_f,
                               mi)
                return m, mi

            m, mi = sweep
            gmax = jnp.max(m)
            cand = jnp.where(m == gmax, mi, jnp.float32(1e9))
            fnew = jnp.min(cand)
            return jnp.full((_L,), fnew, jnp.float32).astype(jnp.int32)

        lax.fori_loop(0, _S, fps_step, fstart)

        def gather_value_row(d, carry):
            pltpu.sync_copy(values2d.at[_D * b + d], rowv)

            @plsc.parallel_loop(0, _SCH, 1, unroll=8)
            def _g16(j):
                sl = pl.ds(j * _L, _L)
                outrow[sl] = plsc.load_gather(rowv, [idx_v[sl]])
            pltpu.sync_copy(outrow, outv2d.at[_D * b + d])
            return carry

        lax.fori_loop(0, _D, gather_value_row, 0)

        for ch, src in enumerate((xv, yv, zv)):
            @plsc.parallel_loop(0, _SCH, 1, unroll=8)
            def _g16c(j, src=src):
                sl = pl.ds(j * _L, _L)
                outrow[sl] = plsc.load_gather(src, [idx_v[sl]])

            pltpu.sync_copy(outrow, outc2d.at[_C * b + ch])


@jax.jit
def _run(coords2d, values2d, f0):
    mesh = plsc.VectorSubcoreMesh(core_axis_name="c", subcore_axis_name="s")
    return pl.kernel(
        _fps_body,
        out_type=(
            jax.ShapeDtypeStruct((_B * _C, _S), jnp.float32),
            jax.ShapeDtypeStruct((_B * _D, _S), jnp.float32),
        ),
        mesh=mesh,
        scratch_types=[
            pltpu.VMEM((_N,), jnp.float32),
            pltpu.VMEM((_N,), jnp.float32),
            pltpu.VMEM((_N,), jnp.float32),
            pltpu.VMEM((_N,), jnp.float32),
            pltpu.VMEM((_S,), jnp.int32),
            pltpu.VMEM((_L,), jnp.float32),
            pltpu.VMEM((_N,), jnp.float32),
            pltpu.VMEM((_S,), jnp.float32),
        ],
        compiler_params=pltpu.CompilerParams(needs_layout_passes=False),
        name="fps_subsample_sc",
    )(coords2d, values2d, f0)


def kernel(coords, values):
    B, C, N = coords.shape
    _, D, _ = values.shape
    S = _S
    f0 = jax.random.randint(jax.random.key(42), (B,), 0, N).astype(jnp.float32)
    f0 = jnp.broadcast_to(f0[:, None], (B, 16))
    outc2d, outv2d = _run(coords.reshape(B * C, N),
                          values.reshape(B * D, N), f0)
    return outc2d.reshape(B, C, S), outv2d.reshape(B, D, S)


# speedup vs baseline: 27.5992x; 1.3178x over previous
import jax
import jax.numpy as jnp
from jax import lax
from jax.experimental import pallas as pl
from jax.experimental.pallas import tpu as pltpu
from jax.experimental.pallas import tpu_sc as plsc

_B = 16
_N = 4096
_S = 2048
_C = 3
_D = 128
_L = 16
_H = _N // 2
_HCH = _H // _L
_SCH = _S // _L


def _fps_body(coords2d, values2d, f0_hbm, outc2d, outv2d,
              xv, yv, zv, dist_v, idx_v, f0v, rowv, outrow,
              msg_out, msg_in, shared):
    c = lax.axis_index("c")
    s = lax.axis_index("s")
    b = c * 8 + s // 2
    h = s % 2
    lane = lax.iota(jnp.int32, _L)
    lane_f = lane.astype(jnp.float32)
    hoff = h * _H
    hoff_f = hoff.astype(jnp.float32)
    partner_is_first = (jnp.full((_L,), h, jnp.int32) == 1)

    pltpu.sync_copy(coords2d.at[_C * b + 0], xv)
    pltpu.sync_copy(coords2d.at[_C * b + 1], yv)
    pltpu.sync_copy(coords2d.at[_C * b + 2], zv)
    pltpu.sync_copy(f0_hbm.at[b], f0v)
    fstart = f0v[...].astype(jnp.int32)

    @plsc.parallel_loop(0, _HCH, 1, unroll=8)
    def _init(j):
        dist_v[pl.ds(j * _L, _L)] = jnp.full((_L,), 1e10, jnp.float32)

    def fps_step(i, fvec):
        cx = plsc.load_gather(xv, [fvec])
        cy = plsc.load_gather(yv, [fvec])
        cz = plsc.load_gather(zv, [fvec])
        plsc.store_scatter(idx_v, [jnp.full((_L,), i, jnp.int32)],
                           fvec, mask=lane == 0)

        acc0 = (jnp.full((_L,), -1.0, jnp.float32),
                jnp.full((_L,), 0.0, jnp.float32))

        @plsc.parallel_loop(0, _HCH, 1, unroll=8, carry=acc0)
        def sweep(j, carry):
            m, mi = carry
            sl = pl.ds(j * _L, _L)
            gl = pl.ds(hoff + j * _L, _L)
            dx = xv[gl] - cx
            dy = yv[gl] - cy
            dz = zv[gl] - cz
            d = (dx * dx + dz * dz) + dy * dy
            dd = dist_v[sl]
            dn = jnp.where(d < dd, d, dd)
            dist_v[sl] = dn
            cond = dn > m
            m = jnp.where(cond, dn, m)
            mi = jnp.where(
                cond, ((j * _L).astype(jnp.float32) + hoff_f) + lane_f, mi)
            return m, mi

        m, mi = sweep
        gmax = jnp.max(m)
        cand = jnp.where(m == gmax, mi, jnp.float32(1e9))
        fidx = jnp.min(cand)
        msg_out[pl.ds(0, _L)] = jnp.full((_L,), gmax, jnp.float32)
        msg_out[pl.ds(_L, _L)] = jnp.full((_L,), fidx, jnp.float32)

        par = jnp.bitwise_and(i, 1)
        pltpu.sync_copy(msg_out, shared.at[pl.ds((s + 16 * par) * (2 * _L), 2 * _L)])
        plsc.subcore_barrier()
        pltpu.sync_copy(shared.at[pl.ds(((s ^ 1) + 16 * par) * (2 * _L), 2 * _L)], msg_in)
        pm = msg_in[pl.ds(0, _L)]
        pi = msg_in[pl.ds(_L, _L)]
        mym = jnp.full((_L,), gmax, jnp.float32)
        myi = jnp.full((_L,), fidx, jnp.float32)
        pwin = (pm > mym) | ((pm == mym) & partner_is_first)
        return jnp.where(pwin, pi, myi).astype(jnp.int32)

    lax.fori_loop(0, _S, fps_step, fstart)

    def gather_value_row(d, carry):
        pltpu.sync_copy(values2d.at[_D * b + 64 * h + d], rowv)

        @plsc.parallel_loop(0, _SCH, 1, unroll=8)
        def _g16(j):
            sl = pl.ds(j * _L, _L)
            outrow[sl] = plsc.load_gather(rowv, [idx_v[sl]])

        pltpu.sync_copy(outrow, outv2d.at[_D * b + 64 * h + d])
        return carry

    lax.fori_loop(0, 64, gather_value_row, 0)

    for ch, src in enumerate((xv, yv, zv)):
        @pl.when(h == (0 if ch < 2 else 1))
        def _(src=src, ch=ch):
            @plsc.parallel_loop(0, _SCH, 1, unroll=8)
            def _g16c(j):
                sl = pl.ds(j * _L, _L)
                outrow[sl] = plsc.load_gather(src, [idx_v[sl]])

            pltpu.sync_copy(outrow, outc2d.at[_C * b + ch])


@jax.jit
def _run(coords2d, values2d, f0):
    mesh = plsc.VectorSubcoreMesh(core_axis_name="c", subcore_axis_name="s")
    return pl.kernel(
        _fps_body,
        out_type=(
            jax.ShapeDtypeStruct((_B * _C, _S), jnp.float32),
            jax.ShapeDtypeStruct((_B * _D, _S), jnp.float32),
        ),
        mesh=mesh,
        scratch_types=[
            pltpu.VMEM((_N,), jnp.float32),
            pltpu.VMEM((_N,), jnp.float32),
            pltpu.VMEM((_N,), jnp.float32),
            pltpu.VMEM((_H,), jnp.float32),
            pltpu.VMEM((_S,), jnp.int32),
            pltpu.VMEM((_L,), jnp.float32),
            pltpu.VMEM((_N,), jnp.float32),
            pltpu.VMEM((_S,), jnp.float32),
            pltpu.VMEM((2 * _L,), jnp.float32),
            pltpu.VMEM((2 * _L,), jnp.float32),
            pltpu.VMEM_SHARED((32 * 2 * _L,), jnp.float32),
        ],
        compiler_params=pltpu.CompilerParams(needs_layout_passes=False),
        name="fps_subsample_sc",
    )(coords2d, values2d, f0)


def kernel(coords, values):
    B, C, N = coords.shape
    _, D, _ = values.shape
    f0 = jax.random.randint(jax.random.key(42), (B,), 0, N).astype(jnp.float32)
    f0 = jnp.broadcast_to(f0[:, None], (B, 16))
    outc2d, outv2d = _run(coords.reshape(B * C, N),
                          values.reshape(B * D, N), f0)
    return outc2d.reshape(B, C, _S), outv2d.reshape(B, D, _S)
